# baseline (speedup 1.0000x reference)
import functools

import jax
import jax.numpy as jnp
from jax import lax
from jax.experimental import pallas as pl
from jax.experimental.pallas import tpu as pltpu
from jax.experimental.pallas import tpu_sc as plsc

_B = 4096
_L = 200
_D = 64
_ASP = 14
_LANES = 128

_NC = 2
_NS = 16
_NW = _NC * _NS
_RPW = _B // _NW
_L2 = _L // 2
_VREGS = _D // 16


def _pool_body(x_hbm, table_hbm, out_hbm, idx_v, buf0, buf1, pooled_v, sem):
    c = lax.axis_index("c")
    s = lax.axis_index("s")
    wid = s * _NC + c
    pltpu.sync_copy(x_hbm.at[pl.ds(wid * (2 * _RPW), 2 * _RPW)], idx_v)

    def row_body(i, carry):
        pltpu.async_copy(table_hbm.at[idx_v.at[2 * i]], buf0, sem).wait()
        pltpu.async_copy(table_hbm.at[idx_v.at[2 * i + 1]], buf1, sem).wait()

        def jbody(j, accs):
            return tuple(
                accs[k] + buf0[j, pl.ds(k * 16, 16)] + buf1[j, pl.ds(k * 16, 16)]
                for k in range(_VREGS))

        accs = lax.fori_loop(
            0, _L2, jbody,
            tuple(jnp.zeros((16,), jnp.float32) for _ in range(_VREGS)))
        for k in range(_VREGS):
            pooled_v[i, pl.ds(k * 16, 16)] = accs[k]
        return carry

    lax.fori_loop(0, _RPW, row_body, 0)
    pltpu.sync_copy(pooled_v, out_hbm.at[pl.ds(wid * _RPW, _RPW)])


@jax.jit
def _pool(x2, table):
    mesh = plsc.VectorSubcoreMesh(core_axis_name="c", subcore_axis_name="s",
                                  num_cores=_NC)
    return pl.kernel(
        _pool_body,
        mesh=mesh,
        compiler_params=pltpu.CompilerParams(use_tc_tiling_on_sc=False),
        out_type=jax.ShapeDtypeStruct((_B, _D), jnp.float32),
        scratch_types=[
            pltpu.VMEM((2 * _RPW, _L2), jnp.int32),
            pltpu.VMEM((_L2, _D), jnp.float32),
            pltpu.VMEM((_L2, _D), jnp.float32),
            pltpu.VMEM((_RPW, _D), jnp.float32),
            pltpu.SemaphoreType.DMA,
        ],
    )(x2, table)


def _head_body(pooled_ref, len_ref, w_ref, b_ref, o_ref):
    p = pooled_ref[...] / len_ref[...]
    logits = jnp.dot(p, w_ref[...], preferred_element_type=jnp.float32)
    logits = logits + b_ref[...]
    m = jnp.max(logits, axis=-1, keepdims=True)
    e = jnp.exp(logits - m)
    o_ref[...] = e / jnp.sum(e, axis=-1, keepdims=True)


@jax.jit
def _head(pooled, lens, w_pad, b_pad):
    return pl.pallas_call(
        _head_body,
        out_shape=jax.ShapeDtypeStruct((_B, _LANES), jnp.float32),
    )(pooled, lens, w_pad, b_pad)


def kernel(x, x_len, table, W, b):
    x2 = x.astype(jnp.int32).reshape(2 * _B, _L2)
    pooled = _pool(x2, table)
    lens = x_len.astype(jnp.float32).reshape(_B, 1)
    w_pad = jnp.pad(W, ((0, 0), (0, _LANES - _ASP)))
    b_pad = jnp.concatenate(
        [b, jnp.full((_LANES - _ASP,), -1e30, jnp.float32)]).reshape(1, _LANES)
    out = _head(pooled, lens, w_pad, b_pad)
    return out[:, :_ASP]


# speedup vs baseline: 1.2420x; 1.2420x over previous
import jax
import jax.numpy as jnp
from jax import lax
from jax.experimental import pallas as pl
from jax.experimental.pallas import tpu as pltpu
from jax.experimental.pallas import tpu_sc as plsc

_B = 4096
_L = 200
_D = 64
_ASP = 14
_LANES = 128

_NC = 2
_NS = 16
_NW = _NC * _NS
_RPW = _B // _NW
_VREGS = _D // 16
_UNROLL = 2


def _pool_body(x_hbm, table_hbm, out_hbm, idx_v, buf0, buf1, pooled_v,
               sem0, sem1):
    c = lax.axis_index("c")
    s = lax.axis_index("s")
    wid = s * _NC + c
    base = wid * _RPW
    pltpu.sync_copy(x_hbm.at[pl.ds(base, _RPW)], idx_v)

    def start(r, buf, sem):
        pltpu.async_copy(table_hbm.at[idx_v.at[r]], buf, sem)

    def wait(r, buf, sem):
        pltpu.make_async_copy(table_hbm.at[idx_v.at[r]], buf, sem).wait()

    def rowsum(buf, r):
        def jbody(j, accs):
            out = []
            for u in range(_UNROLL):
                for k in range(_VREGS):
                    out.append(accs[u * _VREGS + k]
                               + buf[_UNROLL * j + u, pl.ds(k * 16, 16)])
            return tuple(out)

        accs = lax.fori_loop(
            0, _L // _UNROLL, jbody,
            tuple(jnp.zeros((16,), jnp.float32)
                  for _ in range(_UNROLL * _VREGS)))
        for k in range(_VREGS):
            tot = accs[k]
            for u in range(1, _UNROLL):
                tot = tot + accs[u * _VREGS + k]
            pooled_v[r, pl.ds(k * 16, 16)] = tot

    start(0, buf0, sem0)

    def pair_body(i, carry):
        r0 = 2 * i
        r1 = r0 + 1
        start(r1, buf1, sem1)
        wait(r0, buf0, sem0)
        rowsum(buf0, r0)

        @pl.when(i < _RPW // 2 - 1)
        def _():
            start(r0 + 2, buf0, sem0)

        wait(r1, buf1, sem1)
        rowsum(buf1, r1)
        return carry

    lax.fori_loop(0, _RPW // 2, pair_body, 0)
    pltpu.sync_copy(pooled_v, out_hbm.at[pl.ds(base, _RPW)])


@jax.jit
def _pool(x, table):
    mesh = plsc.VectorSubcoreMesh(core_axis_name="c", subcore_axis_name="s",
                                  num_cores=_NC)
    return pl.kernel(
        _pool_body,
        mesh=mesh,
        compiler_params=pltpu.CompilerParams(use_tc_tiling_on_sc=False),
        out_type=jax.ShapeDtypeStruct((_B, _D), jnp.float32),
        scratch_types=[
            pltpu.VMEM((_RPW, _L), jnp.int32),
            pltpu.VMEM((_L, _D), jnp.float32),
            pltpu.VMEM((_L, _D), jnp.float32),
            pltpu.VMEM((_RPW, _D), jnp.float32),
            pltpu.SemaphoreType.DMA,
            pltpu.SemaphoreType.DMA,
        ],
    )(x, table)


def _head_body(pooled_ref, len_ref, w_ref, b_ref, o_ref):
    p = pooled_ref[...] / len_ref[...]
    logits = jnp.dot(p, w_ref[...], preferred_element_type=jnp.float32)
    logits = logits + b_ref[...]
    m = jnp.max(logits, axis=-1, keepdims=True)
    e = jnp.exp(logits - m)
    o_ref[...] = e / jnp.sum(e, axis=-1, keepdims=True)


@jax.jit
def _head(pooled, lens, w_pad, b_pad):
    return pl.pallas_call(
        _head_body,
        out_shape=jax.ShapeDtypeStruct((_B, _LANES), jnp.float32),
    )(pooled, lens, w_pad, b_pad)


def kernel(x, x_len, table, W, b):
    pooled = _pool(x.astype(jnp.int32), table)
    lens = x_len.astype(jnp.float32).reshape(_B, 1)
    w_pad = jnp.pad(W, ((0, 0), (0, _LANES - _ASP)))
    b_pad = jnp.concatenate(
        [b, jnp.full((_LANES - _ASP,), -1e30, jnp.float32)]).reshape(1, _LANES)
    out = _head(pooled, lens, w_pad, b_pad)
    return out[:, :_ASP]
